# baseline (speedup 1.0000x reference)
import jax
import jax.numpy as jnp
from jax.experimental import pallas as pl
from jax.experimental.pallas import tpu as pltpu


def _cbsq_body(cb_ref, out_ref):
    cb = cb_ref[...]
    out_ref[...] = jnp.sum(cb * cb, axis=1)[None, :]


def _nn_body(emb_ref, cb_ref, cbsq_ref, out_ref):
    bn = emb_ref.shape[0]
    k = cb_ref.shape[0]
    rb = 64

    emb = emb_ref[...]
    emb2 = emb * -2.0
    emb_sq = jnp.sum(emb * emb, axis=1, keepdims=True)
    prod = jax.lax.dot_general(
        emb2, cb_ref[...], (((1,), (1,)), ((), ())),
        preferred_element_type=jnp.float32)
    cbsq = cbsq_ref[...]

    lane = jax.lax.broadcasted_iota(jnp.int32, (rb, 128), 1)
    for r in range(bn // rb):
        rs = slice(r * rb, (r + 1) * rb)
        esq = jnp.broadcast_to(emb_sq[rs], (rb, 128))
        mval = jnp.full((rb, 128), jnp.inf, jnp.float32)
        midx = jnp.zeros((rb, 128), jnp.int32)
        for j in range(k // 128):
            sl = slice(j * 128, (j + 1) * 128)
            d2 = (esq + cbsq[:, sl]) + prod[rs, sl]
            upd = d2 < mval
            mval = jnp.where(upd, d2, mval)
            midx = jnp.where(upd, lane + (j * 128), midx)
        m = jnp.min(mval, axis=1, keepdims=True)
        cand = jnp.where(mval == m, midx, k)
        out_ref[0, 0, rs] = jnp.min(cand, axis=1)


def kernel(emb, codebook):
    n, d = emb.shape
    k = codebook.shape[0]
    bn = 256
    g = n // bn
    cbsq = pl.pallas_call(
        _cbsq_body,
        out_shape=jax.ShapeDtypeStruct((1, k), jnp.float32),
    )(codebook)
    idx = pl.pallas_call(
        _nn_body,
        grid=(g,),
        in_specs=[
            pl.BlockSpec((bn, d), lambda i: (i, 0)),
            pl.BlockSpec((k, d), lambda i: (0, 0)),
            pl.BlockSpec((1, k), lambda i: (0, 0)),
        ],
        out_specs=pl.BlockSpec((1, 1, bn), lambda i: (i, 0, 0)),
        out_shape=jax.ShapeDtypeStruct((g, 1, bn), jnp.int32),
        compiler_params=pltpu.CompilerParams(
            dimension_semantics=("parallel",)),
    )(emb, codebook, cbsq)
    idx = idx.reshape(n).astype(jnp.int64)
    attn = jnp.ones_like(idx)
    return idx, attn


# speedup vs baseline: 1.1533x; 1.1533x over previous
import jax
import jax.numpy as jnp
from jax.experimental import pallas as pl
from jax.experimental.pallas import tpu as pltpu


def _nn_body(emb_ref, cb_ref, out_ref, cbsq_ref):
    bn = emb_ref.shape[0]
    k = cb_ref.shape[0]
    rb = 64

    @pl.when(pl.program_id(0) == 0)
    def _():
        cb = cb_ref[...]
        cbsq_ref[...] = jnp.sum(cb * cb, axis=1)[None, :]

    emb = emb_ref[...]
    emb2 = emb * -2.0
    emb_sq = jnp.sum(emb * emb, axis=1, keepdims=True)
    prod = jax.lax.dot_general(
        emb2, cb_ref[...], (((1,), (1,)), ((), ())),
        preferred_element_type=jnp.float32)
    cbsq = cbsq_ref[...]

    lane = jax.lax.broadcasted_iota(jnp.int32, (rb, 128), 1)
    for r in range(bn // rb):
        rs = slice(r * rb, (r + 1) * rb)
        esq = jnp.broadcast_to(emb_sq[rs], (rb, 128))
        mval = jnp.full((rb, 128), jnp.inf, jnp.float32)
        midx = jnp.zeros((rb, 128), jnp.int32)
        for j in range(k // 128):
            sl = slice(j * 128, (j + 1) * 128)
            d2 = (esq + cbsq[:, sl]) + prod[rs, sl]
            upd = d2 < mval
            mval = jnp.where(upd, d2, mval)
            midx = jnp.where(upd, lane + (j * 128), midx)
        m = jnp.min(mval, axis=1, keepdims=True)
        cand = jnp.where(mval == m, midx, k)
        out_ref[0, 0, rs] = jnp.min(cand, axis=1)


def kernel(emb, codebook):
    n, d = emb.shape
    k = codebook.shape[0]
    bn = 512
    g = n // bn
    idx = pl.pallas_call(
        _nn_body,
        grid=(g,),
        in_specs=[
            pl.BlockSpec((bn, d), lambda i: (i, 0)),
            pl.BlockSpec((k, d), lambda i: (0, 0)),
        ],
        out_specs=pl.BlockSpec((1, 1, bn), lambda i: (i, 0, 0)),
        out_shape=jax.ShapeDtypeStruct((g, 1, bn), jnp.int32),
        scratch_shapes=[pltpu.VMEM((1, k), jnp.float32)],
        compiler_params=pltpu.CompilerParams(
            dimension_semantics=("arbitrary",)),
    )(emb, codebook)
    idx = idx.reshape(n).astype(jnp.int64)
    attn = jnp.ones_like(idx)
    return idx, attn
